# baseline (speedup 1.0000x reference)
import jax
import jax.numpy as jnp
from jax.experimental import pallas as pl
from jax.experimental.pallas import tpu as pltpu

BN_EPS = 1e-5
LEAKY_SLOPE = 0.01
KS = 3


def _build_patches(x_ref, patches_ref, Cin, H, W):
    HW = H * W
    col = jax.lax.broadcasted_iota(jnp.int32, (1, HW), 1)
    xb = x_ref[0].reshape(Cin, HW).astype(jnp.bfloat16)
    zero = jnp.bfloat16(0.0)
    for kw in range(KS):
        if kw == 0:
            base = jnp.where((col % W) != (W - 1), xb, zero)
        elif kw == 2:
            base = jnp.where((col % W) != 0, xb, zero)
        else:
            base = xb
        for kh in range(KS):
            off = (kh - 1) * W + (kw - 1)
            r0 = (kh * KS + kw) * Cin
            s = off % HW
            if s:
                rolled = jnp.concatenate([base[:, s:], base[:, :s]], axis=1)
            else:
                rolled = base
            patches_ref[r0:r0 + Cin, :] = rolled
            if off < 0:
                patches_ref[r0:r0 + Cin, 0:-off] = jnp.zeros(
                    (Cin, -off), jnp.bfloat16)
            elif off > 0:
                patches_ref[r0:r0 + Cin, HW - off:HW] = jnp.zeros(
                    (Cin, off), jnp.bfloat16)


def kernel(x_nchw, weight_oihw, bias, gamma, beta):
    del bias

    N, Cin, H, W = x_nchw.shape
    Cout = weight_oihw.shape[0]
    HW = H * W
    KK = KS * KS
    inv_count = 1.0 / float(N * HW)

    w_flat = jnp.transpose(weight_oihw, (0, 2, 3, 1)).reshape(
        Cout, KK * Cin).astype(jnp.bfloat16)
    gb = jnp.stack([gamma, beta], axis=0).astype(jnp.float32)

    def body(w_ref, gb_ref, x_ref, o_ref,
             patches_ref, conv_ref, acc_ref, par_ref):
        p = pl.program_id(0)
        n = pl.program_id(1)

        @pl.when(p == 0)
        def _():
            _build_patches(x_ref, patches_ref, Cin, H, W)
            conv = jnp.dot(w_ref[...], patches_ref[...],
                           preferred_element_type=jnp.float32)
            conv_ref[n] = conv
            partial = jnp.concatenate(
                [jnp.sum(conv, axis=1, keepdims=True),
                 jnp.sum(conv * conv, axis=1, keepdims=True)], axis=1)

            @pl.when(n == 0)
            def _():
                acc_ref[...] = jnp.zeros_like(acc_ref)

            acc_ref[...] = acc_ref[...] + partial

        @pl.when((p == 1) & (n == 0))
        def _():
            mean = acc_ref[:, 0:1] * inv_count
            var = jnp.maximum(acc_ref[:, 1:2] * inv_count - mean * mean, 0.0)
            scale = gb_ref[0:1, :].T * jax.lax.rsqrt(var + BN_EPS)
            shift = gb_ref[1:2, :].T - mean * scale
            par_ref[...] = jnp.concatenate([scale, shift], axis=1)

        @pl.when(p == 1)
        def _():
            y = (conv_ref[n] * par_ref[:, 0:1]
                 + par_ref[:, 1:2])
            y = jnp.maximum(y, LEAKY_SLOPE * y).astype(o_ref.dtype)
            o_ref[0] = y.reshape(Cout, H, W)

    out = pl.pallas_call(
        body,
        grid=(2, N),
        in_specs=[
            pl.BlockSpec((Cout, KK * Cin), lambda p, n: (0, 0)),
            pl.BlockSpec((2, Cout), lambda p, n: (0, 0)),
            pl.BlockSpec((1, Cin, H, W), lambda p, n: (n * (1 - p), 0, 0, 0)),
        ],
        out_specs=pl.BlockSpec((1, Cout, H, W), lambda p, n: (n * p, 0, 0, 0)),
        out_shape=jax.ShapeDtypeStruct((N, Cout, H, W), x_nchw.dtype),
        scratch_shapes=[
            pltpu.VMEM((KK * Cin, HW), jnp.bfloat16),
            pltpu.VMEM((N, Cout, HW), jnp.float32),
            pltpu.VMEM((Cout, 2), jnp.float32),
            pltpu.VMEM((Cout, 2), jnp.float32),
        ],
        compiler_params=pltpu.CompilerParams(
            dimension_semantics=("arbitrary", "arbitrary"),
            vmem_limit_bytes=60 * 1024 * 1024,
        ),
    )(w_flat, gb, x_nchw)

    return out


# speedup vs baseline: 1.3879x; 1.3879x over previous
import jax
import jax.numpy as jnp
from jax.experimental import pallas as pl
from jax.experimental.pallas import tpu as pltpu

BN_EPS = 1e-5
LEAKY_SLOPE = 0.01
KS = 3


def _build_patches(x_ref, patches_ref, Cin, H, W):
    HW = H * W
    col = jax.lax.broadcasted_iota(jnp.int32, (1, HW), 1)
    xb = x_ref[0].reshape(Cin, HW).astype(jnp.bfloat16)
    zero = jnp.bfloat16(0.0)
    for kw in range(KS):
        if kw == 0:
            base = jnp.where((col % W) != (W - 1), xb, zero)
        elif kw == 2:
            base = jnp.where((col % W) != 0, xb, zero)
        else:
            base = xb
        for kh in range(KS):
            off = (kh - 1) * W + (kw - 1)
            r0 = (kh * KS + kw) * Cin
            s = off % HW
            if s:
                rolled = jnp.concatenate([base[:, s:], base[:, :s]], axis=1)
            else:
                rolled = base
            patches_ref[r0:r0 + Cin, :] = rolled
            if off < 0:
                patches_ref[r0:r0 + Cin, 0:-off] = jnp.zeros(
                    (Cin, -off), jnp.bfloat16)
            elif off > 0:
                patches_ref[r0:r0 + Cin, HW - off:HW] = jnp.zeros(
                    (Cin, off), jnp.bfloat16)


def kernel(x_nchw, weight_oihw, bias, gamma, beta):
    del bias

    N, Cin, H, W = x_nchw.shape
    Cout = weight_oihw.shape[0]
    HW = H * W
    KK = KS * KS
    inv_count = 1.0 / float(N * HW)

    w_flat = jnp.transpose(weight_oihw, (0, 2, 3, 1)).reshape(
        Cout, KK * Cin).astype(jnp.bfloat16)
    gb = jnp.stack([gamma, beta], axis=0).astype(jnp.float32)

    def body(w_ref, gb_ref, x_ref, o_ref,
             patches_ref, conv_ref, acc_ref, par_ref):
        p = pl.program_id(0)
        n = pl.program_id(1)

        @pl.when(p == 0)
        def _():
            _build_patches(x_ref, patches_ref, Cin, H, W)
            conv = jnp.dot(w_ref[...], patches_ref[...],
                           preferred_element_type=jnp.float32)
            conv_ref[n] = conv
            partial = jnp.concatenate(
                [jnp.sum(conv, axis=1, keepdims=True),
                 jnp.sum(conv * conv, axis=1, keepdims=True)], axis=1)

            @pl.when(n == 0)
            def _():
                acc_ref[...] = jnp.zeros_like(acc_ref)

            acc_ref[...] = acc_ref[...] + partial

        @pl.when((p == 1) & (n == 0))
        def _():
            mean = acc_ref[:, 0:1] * inv_count
            var = jnp.maximum(acc_ref[:, 1:2] * inv_count - mean * mean, 0.0)
            scale = gb_ref[0:1, :].T * jax.lax.rsqrt(var + BN_EPS)
            shift = gb_ref[1:2, :].T - mean * scale
            par_ref[...] = jnp.concatenate([scale, shift], axis=1)

        @pl.when(p == 1)
        def _():
            y = (conv_ref[n] * par_ref[:, 0:1]
                 + par_ref[:, 1:2])
            o_ref[0] = jnp.maximum(y, LEAKY_SLOPE * y).astype(o_ref.dtype)

    out = pl.pallas_call(
        body,
        grid=(2, N),
        in_specs=[
            pl.BlockSpec((Cout, KK * Cin), lambda p, n: (0, 0)),
            pl.BlockSpec((2, Cout), lambda p, n: (0, 0)),
            pl.BlockSpec((1, Cin, H, W), lambda p, n: (n * (1 - p), 0, 0, 0)),
        ],
        out_specs=pl.BlockSpec((1, Cout, HW), lambda p, n: (n * p, 0, 0)),
        out_shape=jax.ShapeDtypeStruct((N, Cout, HW), x_nchw.dtype),
        scratch_shapes=[
            pltpu.VMEM((KK * Cin, HW), jnp.bfloat16),
            pltpu.VMEM((N, Cout, HW), jnp.float32),
            pltpu.VMEM((Cout, 2), jnp.float32),
            pltpu.VMEM((Cout, 2), jnp.float32),
        ],
        compiler_params=pltpu.CompilerParams(
            dimension_semantics=("arbitrary", "arbitrary"),
            vmem_limit_bytes=60 * 1024 * 1024,
        ),
    )(w_flat, gb, x_nchw)

    return out.reshape(N, Cout, H, W)
